# baseline (speedup 1.0000x reference)
import functools

import jax
import jax.numpy as jnp
from jax import lax
from jax.experimental import pallas as pl
from jax.experimental.pallas import tpu as pltpu
from jax.experimental.pallas import tpu_sc as plsc

N = 10000
D = 128
H = 128
O = 32

NC = 2
NS = 16
NW = NC * NS


NPAD = 10240


def _make_spmm(width: int, n_edges: int, chunk: int, nbuf: int):
    assert n_edges % (NW * chunk) == 0
    assert chunk % 8 == 0 and chunk <= 128
    epw = n_edges // NW
    nchunks = epw // chunk
    nring = 2 * nbuf
    assert nchunks % nring == 0
    rows_per_s = NPAD // NS

    mesh = plsc.VectorSubcoreMesh(core_axis_name="c", subcore_axis_name="s")

    @functools.partial(
        pl.kernel,
        mesh=mesh,
        out_type=jax.ShapeDtypeStruct((NC, NPAD, width), jnp.float32),
        scratch_types=[
            [pltpu.VMEM((chunk,), jnp.int32) for _ in range(nring)],
            [pltpu.VMEM((chunk,), jnp.int32) for _ in range(nring)],
            [pltpu.VMEM((chunk, width), jnp.float32) for _ in range(nbuf)],
            pltpu.VMEM_SHARED((NPAD, width), jnp.float32),
            pltpu.SemaphoreType.DMA,
            pltpu.SemaphoreType.DMA,
        ],
    )
    def spmm(sup_hbm, eidx_hbm, zero_hbm, out_hbm,
             rcol, rrow, bufs, acc_sh, sem_i, sem_g):
        c = lax.axis_index("c")
        s = lax.axis_index("s")
        wid = s * NC + c
        base = pl.multiple_of(wid * epw, chunk)
        rbase = pl.multiple_of(s * rows_per_s, rows_per_s)

        def idx_fetch(k, u):
            off = pl.multiple_of(
                base + jnp.minimum(k, nchunks - 1) * chunk, chunk)
            pltpu.async_copy(eidx_hbm.at[pl.ds(n_edges + off, chunk)],
                             rcol[u], sem_i)
            pltpu.async_copy(eidx_hbm.at[pl.ds(off, chunk)], rrow[u], sem_i)

        def idx_wait(u):
            pltpu.make_async_copy(eidx_hbm.at[pl.ds(0, chunk)],
                                  rcol[u], sem_i).wait()
            pltpu.make_async_copy(eidx_hbm.at[pl.ds(0, chunk)],
                                  rrow[u], sem_i).wait()

        def gather(k, b, u):
            pltpu.async_copy(sup_hbm.at[rcol[u]], bufs[b], sem_g)

        def gwait(b):
            pltpu.make_async_copy(sup_hbm.at[rcol[0]], bufs[b], sem_g).wait()

        pltpu.sync_copy(zero_hbm, acc_sh.at[pl.ds(rbase, rows_per_s)])

        for u in range(nring):
            idx_fetch(u, u)
        for u in range(nbuf):
            idx_wait(u)
        plsc.subcore_barrier()
        for b in range(nbuf):
            gather(b, b, b)

        def body(g, carry):
            for j in range(nring):
                i = g * nring + j
                b = j % nbuf
                gwait(b)
                pltpu.sync_copy(bufs[b], acc_sh.at[rrow[j]], add=True)
                idx_fetch(i + nring, j)
                idx_wait(j)
                gather(i + nbuf, b, (j + nbuf) % nring)
            return carry

        lax.fori_loop(0, nchunks // nring, body, 0)
        for b in range(nbuf):
            gwait(b)
            idx_wait(b)
        plsc.subcore_barrier()

        pltpu.sync_copy(acc_sh.at[pl.ds(rbase, rows_per_s)],
                        out_hbm.at[c, pl.ds(rbase, rows_per_s)])

    return spmm


_BN = 1000


def _tc1_body(x_ref, w_ref, b_ref, w2_ref, b2_ref, sup_ref, mlp_ref):
    sup = jnp.dot(x_ref[...], w_ref[...], preferred_element_type=jnp.float32)
    sup_ref[...] = sup
    h = jnp.maximum(sup + b_ref[...], 0.0)
    mlp_ref[...] = (
        jnp.dot(h, w2_ref[...], preferred_element_type=jnp.float32) + b2_ref[...]
    )


def _tc2_body(p_ref, b_ref, out_ref):
    out_ref[...] = jnp.maximum(p_ref[0] + p_ref[1] + b_ref[...], 0.0)


def _tc3_body(q_ref, w2_ref, mlp_ref, mw_ref, mean_ref, std_ref):
    g = jnp.dot(q_ref[0] + q_ref[1], w2_ref[...],
                preferred_element_type=jnp.float32)
    mw = mw_ref[0, 0]
    ratio = jax.nn.sigmoid(mw)
    mlp = mlp_ref[...]
    mean_ref[...] = g[:, :O] * mw + mlp[:, :O] * (1.0 - mw)
    std_ref[...] = g[:, O:] * ratio + mlp[:, O:] * (1.0 - ratio)


def _row_block(bn, cols):
    return pl.BlockSpec((bn, cols), lambda i: (i, 0))


def _full(shape):
    return pl.BlockSpec(shape, lambda i: tuple(0 for _ in shape))


def kernel(x, edge_index, mixture_weight, hidden_weight, hidden_bias,
           mean_weight, mean_bias, log_std_weight, log_std_bias):
    n, d = x.shape
    h = hidden_weight.shape[1]
    o = mean_weight.shape[1]
    e = edge_index.shape[1]
    assert n == N and d == D and h == H and o == O

    w2 = jnp.concatenate([mean_weight, log_std_weight], axis=1)
    b2 = jnp.concatenate([mean_bias, log_std_bias])[None, :]
    bias = hidden_bias[None, :]

    grid = (N // _BN,)

    support, mlp_cat = pl.pallas_call(
        _tc1_body,
        grid=grid,
        in_specs=[
            _row_block(_BN, D),
            _full((D, H)),
            _full((1, H)),
            _full((H, 2 * O)),
            _full((1, 2 * O)),
        ],
        out_specs=[_row_block(_BN, H), _row_block(_BN, 2 * O)],
        out_shape=[
            jax.ShapeDtypeStruct((N, H), jnp.float32),
            jax.ShapeDtypeStruct((N, 2 * O), jnp.float32),
        ],
    )(x, hidden_weight, bias, w2, b2)

    chunk, nbuf = 40, 5
    quantum = chunk * 2 * nbuf
    epw = -(-e // (NW * quantum)) * quantum
    epad = NW * epw
    pad = epad - e
    ei = edge_index.astype(jnp.int32)
    pad_iota = jax.lax.iota(jnp.int32, pad)
    eidx = jnp.concatenate([
        ei[0], N + pad_iota % (NPAD - N),
        ei[1], (pad_iota * 37) % N])

    spmm_h = _make_spmm(H, epad, chunk, nbuf)
    zeros_h = jnp.zeros((NPAD // NS, H), jnp.float32)

    p = spmm_h(support, eidx, zeros_h)

    hidden_gcn = pl.pallas_call(
        _tc2_body,
        grid=grid,
        in_specs=[
            pl.BlockSpec((2, _BN, H), lambda i: (0, i, 0)),
            _full((1, H)),
        ],
        out_specs=_row_block(_BN, H),
        out_shape=jax.ShapeDtypeStruct((N, H), jnp.float32),
    )(p, bias)

    q = spmm_h(hidden_gcn, eidx, zeros_h)

    z_mean, z_log_std = pl.pallas_call(
        _tc3_body,
        grid=grid,
        in_specs=[
            pl.BlockSpec((2, _BN, H), lambda i: (0, i, 0)),
            _full((H, 2 * O)),
            _row_block(_BN, 2 * O),
            _full((1, 1)),
        ],
        out_specs=[_row_block(_BN, O), _row_block(_BN, O)],
        out_shape=[
            jax.ShapeDtypeStruct((N, O), jnp.float32),
            jax.ShapeDtypeStruct((N, O), jnp.float32),
        ],
    )(q, w2, mlp_cat, mixture_weight.reshape(1, 1))

    return (z_mean, z_log_std)


# speedup vs baseline: 1.0380x; 1.0380x over previous
import functools

import jax
import jax.numpy as jnp
from jax import lax
from jax.experimental import pallas as pl
from jax.experimental.pallas import tpu as pltpu
from jax.experimental.pallas import tpu_sc as plsc

N = 10000
D = 128
H = 128
O = 32

NC = 2
NS = 16
NW = NC * NS


NPAD = 10240


def _make_spmm(width: int, n_edges: int, chunk: int, nbuf: int):
    assert n_edges % (NW * chunk) == 0
    assert chunk % 8 == 0 and chunk <= 128
    epw = n_edges // NW
    nchunks = epw // chunk
    nring = 2 * nbuf
    assert nchunks % nring == 0
    rows_per_s = NPAD // NS

    mesh = plsc.VectorSubcoreMesh(core_axis_name="c", subcore_axis_name="s")

    @functools.partial(
        pl.kernel,
        mesh=mesh,
        out_type=jax.ShapeDtypeStruct((NC, NPAD, width), jnp.float32),
        scratch_types=[
            [pltpu.VMEM((chunk,), jnp.int32) for _ in range(nring)],
            [pltpu.VMEM((chunk,), jnp.int32) for _ in range(nring)],
            [pltpu.VMEM((chunk, width), jnp.float32) for _ in range(nbuf)],
            pltpu.VMEM_SHARED((NPAD, width), jnp.float32),
            pltpu.SemaphoreType.DMA,
            pltpu.SemaphoreType.DMA,
        ],
    )
    def spmm(sup_hbm, eidx_hbm, zero_hbm, out_hbm,
             rcol, rrow, bufs, acc_sh, sem_i, sem_g):
        c = lax.axis_index("c")
        s = lax.axis_index("s")
        wid = s * NC + c
        base = pl.multiple_of(wid * epw, chunk)
        rbase = pl.multiple_of(s * rows_per_s, rows_per_s)

        def idx_fetch(k, u):
            off = pl.multiple_of(
                base + jnp.minimum(k, nchunks - 1) * chunk, chunk)
            pltpu.async_copy(eidx_hbm.at[pl.ds(n_edges + off, chunk)],
                             rcol[u], sem_i)
            pltpu.async_copy(eidx_hbm.at[pl.ds(off, chunk)], rrow[u], sem_i)

        def idx_wait(u):
            pltpu.make_async_copy(eidx_hbm.at[pl.ds(0, chunk)],
                                  rcol[u], sem_i).wait()
            pltpu.make_async_copy(eidx_hbm.at[pl.ds(0, chunk)],
                                  rrow[u], sem_i).wait()

        def gather(k, b, u):
            pltpu.async_copy(sup_hbm.at[rcol[u]], bufs[b], sem_g)

        def gwait(b):
            pltpu.make_async_copy(sup_hbm.at[rcol[0]], bufs[b], sem_g).wait()

        pltpu.sync_copy(zero_hbm, acc_sh.at[pl.ds(rbase, rows_per_s)])

        for u in range(nring):
            idx_fetch(u, u)
        for u in range(nbuf):
            idx_wait(u)
        plsc.subcore_barrier()
        for b in range(nbuf):
            gather(b, b, b)

        def body(g, carry):
            for j in range(nring):
                i = g * nring + j
                b = j % nbuf
                gwait(b)
                pltpu.sync_copy(bufs[b], acc_sh.at[rrow[j]], add=True)
                idx_fetch(i + nring, j)
                idx_wait(j)
                gather(i + nbuf, b, (j + nbuf) % nring)
            return carry

        lax.fori_loop(0, nchunks // nring, body, 0)
        for b in range(nbuf):
            gwait(b)
            idx_wait(b)
        plsc.subcore_barrier()

        pltpu.sync_copy(acc_sh.at[pl.ds(rbase, rows_per_s)],
                        out_hbm.at[c, pl.ds(rbase, rows_per_s)])

    return spmm


_BN = 1000


def _tc1_body(x_ref, w_ref, b_ref, w2_ref, b2_ref, sup_ref, mlp_ref):
    sup = jnp.dot(x_ref[...], w_ref[...], preferred_element_type=jnp.float32)
    sup_ref[...] = sup
    h = jnp.maximum(sup + b_ref[...], 0.0)
    mlp_ref[...] = (
        jnp.dot(h, w2_ref[...], preferred_element_type=jnp.float32) + b2_ref[...]
    )


def _tc2_body(p_ref, b_ref, out_ref):
    out_ref[...] = jnp.maximum(p_ref[0] + p_ref[1] + b_ref[...], 0.0)


def _tc3_body(q_ref, w2_ref, mlp_ref, mw_ref, mean_ref, std_ref):
    g = jnp.dot(q_ref[0] + q_ref[1], w2_ref[...],
                preferred_element_type=jnp.float32)
    mw = mw_ref[0, 0]
    ratio = jax.nn.sigmoid(mw)
    mlp = mlp_ref[...]
    mean_ref[...] = g[:, :O] * mw + mlp[:, :O] * (1.0 - mw)
    std_ref[...] = g[:, O:] * ratio + mlp[:, O:] * (1.0 - ratio)


def _row_block(bn, cols):
    return pl.BlockSpec((bn, cols), lambda i: (i, 0))


def _full(shape):
    return pl.BlockSpec(shape, lambda i: tuple(0 for _ in shape))


def kernel(x, edge_index, mixture_weight, hidden_weight, hidden_bias,
           mean_weight, mean_bias, log_std_weight, log_std_bias):
    n, d = x.shape
    h = hidden_weight.shape[1]
    o = mean_weight.shape[1]
    e = edge_index.shape[1]
    assert n == N and d == D and h == H and o == O

    w2 = jnp.concatenate([mean_weight, log_std_weight], axis=1)
    b2 = jnp.concatenate([mean_bias, log_std_bias])[None, :]
    bias = hidden_bias[None, :]

    grid = (N // _BN,)

    support, mlp_cat = pl.pallas_call(
        _tc1_body,
        grid=grid,
        in_specs=[
            _row_block(_BN, D),
            _full((D, H)),
            _full((1, H)),
            _full((H, 2 * O)),
            _full((1, 2 * O)),
        ],
        out_specs=[_row_block(_BN, H), _row_block(_BN, 2 * O)],
        out_shape=[
            jax.ShapeDtypeStruct((N, H), jnp.float32),
            jax.ShapeDtypeStruct((N, 2 * O), jnp.float32),
        ],
    )(x, hidden_weight, bias, w2, b2)

    chunk, nbuf = 40, 5
    quantum = chunk * 2 * nbuf
    epw = -(-e // (NW * quantum)) * quantum
    epad = NW * epw
    pad = epad - e
    ei = edge_index.astype(jnp.int32)
    if pad == 0:
        eidx = ei.reshape(2 * e)
    else:
        pad_iota = jax.lax.iota(jnp.int32, pad)
        eidx = jnp.concatenate([
            ei[0], N + pad_iota % (NPAD - N),
            ei[1], (pad_iota * 37) % N])

    spmm_h = _make_spmm(H, epad, chunk, nbuf)
    zeros_h = jnp.zeros((NPAD // NS, H), jnp.float32)

    p = spmm_h(support, eidx, zeros_h)

    hidden_gcn = pl.pallas_call(
        _tc2_body,
        grid=grid,
        in_specs=[
            pl.BlockSpec((2, _BN, H), lambda i: (0, i, 0)),
            _full((1, H)),
        ],
        out_specs=_row_block(_BN, H),
        out_shape=jax.ShapeDtypeStruct((N, H), jnp.float32),
    )(p, bias)

    q = spmm_h(hidden_gcn, eidx, zeros_h)

    z_mean, z_log_std = pl.pallas_call(
        _tc3_body,
        grid=grid,
        in_specs=[
            pl.BlockSpec((2, _BN, H), lambda i: (0, i, 0)),
            _full((H, 2 * O)),
            _row_block(_BN, 2 * O),
            _full((1, 1)),
        ],
        out_specs=[_row_block(_BN, O), _row_block(_BN, O)],
        out_shape=[
            jax.ShapeDtypeStruct((N, O), jnp.float32),
            jax.ShapeDtypeStruct((N, O), jnp.float32),
        ],
    )(q, w2, mlp_cat, mixture_weight.reshape(1, 1))

    return (z_mean, z_log_std)
